# baseline (speedup 1.0000x reference)
import functools

import jax
import jax.numpy as jnp
from jax.experimental import pallas as pl
from jax.experimental.pallas import tpu as pltpu
from jax.experimental.pallas import tpu_sc as plsc

N_WIN = 14
TOPK = 2
NUM_HEADS = 8
SIDE = 5


def _qkvpool_body(x_ref, w_ref, b_ref, p_ref, q_ref, k_ref, v_ref, xr_ref,
                  *, rows, scale):
    C = x_ref.shape[0]
    Wd = x_ref.shape[2]
    rw = q_ref.shape[2]
    rs = q_ref.shape[3]
    j = pl.program_id(1)
    xc = x_ref[...].reshape(C, rows * Wd)
    tn = (((0,), (0,)), ((), ()))
    nn = (((1,), (0,)), ((), ()))
    x_hi = xc.astype(jnp.bfloat16)
    x_lo = (xc - x_hi.astype(jnp.float32)).astype(jnp.bfloat16)
    qkv = (jax.lax.dot_general(x_hi, w_ref[...], tn,
                               preferred_element_type=jnp.float32) +
           jax.lax.dot_general(x_lo, w_ref[...], tn,
                               preferred_element_type=jnp.float32))
    q = (qkv[:, :C] + b_ref[0:1, :]) * scale
    k = qkv[:, C:2 * C] + b_ref[1:2, :]
    v = qkv[:, 2 * C:] + b_ref[2:3, :]
    q_ref[...] = q.astype(jnp.bfloat16).reshape(1, rows, rw, rs, C)
    k_ref[...] = k.astype(jnp.bfloat16).reshape(1, rows, rw, rs, C)
    v_ref[...] = v.astype(jnp.bfloat16).reshape(1, rows, rw, rs, C)
    xsum = jnp.sum(x_ref[...], axis=1)
    part = jax.lax.dot_general(xsum, p_ref[...], nn,
                               preferred_element_type=jnp.float32)

    @pl.when(j == 0)
    def _():
        xr_ref[...] = part.reshape(1, C, 128)

    @pl.when(j != 0)
    def _():
        xr_ref[...] += part.reshape(1, C, 128)


def _routing_body(xr_ref, wq_ref, wk_ref, b_ref, a_ref):
    xr = xr_ref[...]
    qr = jnp.dot(xr, wq_ref[...], preferred_element_type=jnp.float32) + b_ref[0:1, :]
    kr = jnp.dot(xr, wk_ref[...], preferred_element_type=jnp.float32) + b_ref[1:2, :]
    a = jax.lax.dot_general(qr, kr, (((1,), (1,)), ((), ())),
                            preferred_element_type=jnp.float32)
    pad = a_ref.shape[1] - a.shape[1]
    a_ref[...] = jnp.concatenate(
        [a, jnp.full((a.shape[0], pad), -jnp.inf, jnp.float32)], axis=1)


def _make_sc_topk(R2, padc):
    n_work = 28
    rows_per = R2 // n_work
    chunks = padc // 16
    mesh = plsc.VectorSubcoreMesh(core_axis_name="c", subcore_axis_name="s")

    @functools.partial(
        pl.kernel, mesh=mesh,
        out_type=jax.ShapeDtypeStruct((R2, 16), jnp.int32),
        scratch_types=[pltpu.VMEM((padc,), jnp.float32),
                       pltpu.VMEM((16,), jnp.int32)],
    )
    def _sc_topk(a_hbm, idx_hbm, row_v, idx_v):
        wid = jax.lax.axis_index("s") * 2 + jax.lax.axis_index("c")

        @pl.when(wid < n_work)
        def _():
            lanes = jax.lax.iota(jnp.int32, 16)
            big = jnp.int32(1 << 30)
            neg = jnp.float32(-jnp.inf)
            mask15 = jnp.int32(15)

            dnums = jax.lax.GatherDimensionNumbers(
                offset_dims=(), collapsed_slice_dims=(0,), start_index_map=(0,))

            def shuf(v, perm):
                return jax.lax.gather(
                    v, perm[:, None], dnums, (1,),
                    mode=jax.lax.GatherScatterMode.PROMISE_IN_BOUNDS)

            def allmax(v):
                for sh in (1, 2, 4, 8):
                    v = jnp.maximum(v, shuf(v, (lanes + sh) & mask15))
                return v

            def allmin(v):
                for sh in (1, 2, 4, 8):
                    v = jnp.minimum(v, shuf(v, (lanes + sh) & mask15))
                return v

            for t in range(rows_per):
                r = wid * rows_per + t
                pltpu.sync_copy(a_hbm.at[r], row_v)
                m = jnp.full((16,), neg, jnp.float32)
                for c in range(chunks):
                    m = jnp.maximum(m, row_v[pl.ds(c * 16, 16)])
                m1 = allmax(m)
                best = jnp.full((16,), big, jnp.int32)
                for c in range(chunks):
                    ch = row_v[pl.ds(c * 16, 16)]
                    best = jnp.minimum(
                        best, jnp.where(ch == m1, lanes + c * 16, big))
                i1 = allmin(best)
                m2v = jnp.full((16,), neg, jnp.float32)
                for c in range(chunks):
                    ch = row_v[pl.ds(c * 16, 16)]
                    pos = lanes + c * 16
                    m2v = jnp.maximum(m2v, jnp.where(pos == i1, neg, ch))
                m2 = allmax(m2v)
                best2 = jnp.full((16,), big, jnp.int32)
                for c in range(chunks):
                    ch = row_v[pl.ds(c * 16, 16)]
                    pos = lanes + c * 16
                    best2 = jnp.minimum(
                        best2, jnp.where((ch == m2) & (pos != i1), pos, big))
                i2 = allmin(best2)
                idx_v[...] = jnp.where(lanes == 0, i1,
                                       jnp.where(lanes == 1, i2, 0))
                pltpu.sync_copy(idx_v, idx_hbm.at[r])

    return _sc_topk


def _attn_pair_body(idx_ref, q_ref, k0a_ref, k1a_ref, k0b_ref, k1b_ref,
                    v0a_ref, v1a_ref, v0b_ref, v1b_ref, o_ref, *, rt, nh):
    del idx_ref
    C = q_ref.shape[-1]
    rs = q_ref.shape[1]
    hd = C // nh
    nt = (((1,), (1,)), ((), ()))
    nn = (((1,), (0,)), ((), ()))
    for j, (k0_ref, k1_ref, v0_ref, v1_ref) in enumerate(
            [(k0a_ref, k1a_ref, v0a_ref, v1a_ref),
             (k0b_ref, k1b_ref, v0b_ref, v1b_ref)]):
        q = q_ref[0, :, j, :, :].reshape(rt, C)
        k0 = k0_ref[...].reshape(rt, C)
        k1 = k1_ref[...].reshape(rt, C)
        v0 = v0_ref[...].reshape(rt, C)
        v1 = v1_ref[...].reshape(rt, C)
        outs = []
        for h in range(nh):
            sl = slice(h * hd, (h + 1) * hd)
            qh = q[:, sl]
            e0 = jnp.exp(jax.lax.dot_general(
                qh, k0[:, sl], nt, preferred_element_type=jnp.float32))
            e1 = jnp.exp(jax.lax.dot_general(
                qh, k1[:, sl], nt, preferred_element_type=jnp.float32))
            den = (jnp.sum(e0, axis=1, keepdims=True) +
                   jnp.sum(e1, axis=1, keepdims=True))
            o = (jax.lax.dot_general(e0.astype(jnp.bfloat16), v0[:, sl], nn,
                                     preferred_element_type=jnp.float32) +
                 jax.lax.dot_general(e1.astype(jnp.bfloat16), v1[:, sl], nn,
                                     preferred_element_type=jnp.float32))
            outs.append(o / den)
        o = jnp.concatenate(outs, axis=1).astype(jnp.bfloat16)
        o_ref[0, :, j, :, :] = o.reshape(rs, rs, C)


def _final_body(attn_ref, vc_ref, vp_ref, vn_ref, lw_ref, lb_ref, ow_ref, ob_ref,
                o_ref, sh_ref, *, R, Wd):
    i = pl.program_id(0)
    n = pl.num_programs(0)
    C = vc_ref.shape[-1]
    pe = SIDE // 2
    f32 = jnp.float32
    top = vp_ref[...].astype(f32) * jnp.where(i == 0, 0.0, 1.0)
    bot = vn_ref[...].astype(f32) * jnp.where(i == n - 1, 0.0, 1.0)
    rows = jnp.concatenate([top, vc_ref[...].astype(f32), bot], axis=0)
    zc = jnp.zeros((R + 2 * pe, pe, C), f32)
    padded = jnp.concatenate([zc, rows, zc], axis=1)
    for dx in range(SIDE):
        sh_ref[dx] = padded[:, dx:dx + Wd, :]
    acc = attn_ref[...].astype(f32) + lb_ref[...]
    for dy in range(SIDE):
        for dx in range(SIDE):
            w = lw_ref[dy * SIDE + dx:dy * SIDE + dx + 1, :]
            acc = acc + sh_ref[dx, dy:dy + R] * w
    a2 = acc.reshape(R * Wd, C)
    a_hi = a2.astype(jnp.bfloat16)
    a_lo = (a2 - a_hi.astype(f32)).astype(jnp.bfloat16)
    y = (jnp.dot(a_hi, ow_ref[...], preferred_element_type=f32) +
         jnp.dot(a_lo, ow_ref[...], preferred_element_type=f32)) + ob_ref[...]
    o_ref[...] = y


def kernel(x, qkv_w, qkv_b, lepe_w, lepe_b, out_w, out_b):
    _, C, Hh, Ww = x.shape
    rs = max(1, Hh // N_WIN)
    rh, rw = Hh // rs, Ww // rs
    R2 = rh * rw
    rt = rs * rs
    HW = Hh * Ww
    scale = C ** (-0.5)
    f32 = jnp.float32
    bf16 = jnp.bfloat16

    x3 = x.reshape(C, Hh, Ww)
    w_all = qkv_w.T.astype(bf16)
    wq = qkv_w[0:C].T * (1.0 / rt)
    wk = qkv_w[C:2 * C].T * (1.0 / rt)
    b3 = qkv_b.reshape(3, C)
    rows = rs // 2
    col = jnp.arange(Ww)[:, None]
    pmat = (col // rs == jnp.arange(128)[None, :]).astype(f32)

    blk5 = (1, rs, 1, rs, C)
    cmap2 = lambda *_: (0, 0)

    q5, k5, v5, xrp = pl.pallas_call(
        functools.partial(_qkvpool_body, rows=rows, scale=scale),
        grid=(rh, rs // rows),
        in_specs=[pl.BlockSpec((C, rows, Ww), lambda i, j: (0, 2 * i + j, 0)),
                  pl.BlockSpec((C, 3 * C), cmap2),
                  pl.BlockSpec((3, C), cmap2),
                  pl.BlockSpec((Ww, 128), cmap2)],
        out_specs=[pl.BlockSpec((1, rows, rw, rs, C),
                                lambda i, j: (i, j, 0, 0, 0))] * 3 +
                  [pl.BlockSpec((1, C, 128), lambda i, j: (i, 0, 0))],
        out_shape=[jax.ShapeDtypeStruct((rh, rs, rw, rs, C), bf16)] * 3 +
                  [jax.ShapeDtypeStruct((rh, C, 128), f32)],
    )(x3, w_all, b3, pmat)
    xr = jnp.transpose(xrp[:, :, :rw], (0, 2, 1)).reshape(R2, C)

    padc = ((R2 + 15) // 16) * 16
    a_pad = pl.pallas_call(
        _routing_body,
        out_shape=jax.ShapeDtypeStruct((R2, padc), f32),
    )(xr, wq, wk, b3)
    idx16 = _make_sc_topk(R2, padc)(a_pad)
    idx_flat = idx16[:, :TOPK].reshape(-1)

    rw2 = rw // 2
    blk5p = (1, rs, 2, rs, C)
    qmap = lambda g, idx: (g // rw2, 0, g % rw2, 0, 0)

    def gmap(o):
        return lambda g, idx: (idx[4 * g + o] // rw, 0, idx[4 * g + o] % rw, 0, 0)

    attn5 = pl.pallas_call(
        functools.partial(_attn_pair_body, rt=rt, nh=NUM_HEADS),
        grid_spec=pltpu.PrefetchScalarGridSpec(
            num_scalar_prefetch=1,
            grid=(R2 // 2,),
            in_specs=[pl.BlockSpec(blk5p, qmap)] +
                     [pl.BlockSpec(blk5, gmap(o)) for o in (0, 1, 2, 3)] +
                     [pl.BlockSpec(blk5, gmap(o)) for o in (0, 1, 2, 3)],
            out_specs=pl.BlockSpec(blk5p, qmap),
        ),
        out_shape=jax.ShapeDtypeStruct((rh, rs, rw, rs, C), bf16),
    )(idx_flat, q5, k5, k5, k5, k5, v5, v5, v5, v5)

    R = 4
    attn3 = attn5.reshape(Hh, Ww, C)
    v3 = v5.reshape(Hh, Ww, C)
    lw = lepe_w.reshape(C, SIDE * SIDE).T
    out_t = pl.pallas_call(
        functools.partial(_final_body, R=R, Wd=Ww),
        grid=(Hh // R,),
        in_specs=[pl.BlockSpec((R, Ww, C), lambda i: (i, 0, 0)),
                  pl.BlockSpec((R, Ww, C), lambda i: (i, 0, 0)),
                  pl.BlockSpec((2, Ww, C),
                               lambda i: (jnp.maximum(2 * i - 1, 0), 0, 0)),
                  pl.BlockSpec((2, Ww, C),
                               lambda i: (jnp.minimum(2 * i + 2, Hh // 2 - 1), 0, 0)),
                  pl.BlockSpec((SIDE * SIDE, C), cmap2),
                  pl.BlockSpec((1, C), cmap2),
                  pl.BlockSpec((C, C), cmap2),
                  pl.BlockSpec((1, C), cmap2)],
        out_specs=pl.BlockSpec((R * Ww, C), lambda i: (i, 0)),
        out_shape=jax.ShapeDtypeStruct((HW, C), f32),
        scratch_shapes=[pltpu.VMEM((SIDE, R + 2 * (SIDE // 2), Ww, C), f32)],
    )(attn3, v3, v3, v3, lw, lepe_b.reshape(1, C),
      out_w.T.astype(bf16), out_b.reshape(1, C))

    return out_t.T.reshape(1, C, Hh, Ww)


# speedup vs baseline: 1.1374x; 1.1374x over previous
import functools

import jax
import jax.numpy as jnp
from jax.experimental import pallas as pl
from jax.experimental.pallas import tpu as pltpu
from jax.experimental.pallas import tpu_sc as plsc

N_WIN = 14
TOPK = 2
NUM_HEADS = 8
SIDE = 5


def _qkvpool_body(x_ref, w_ref, b_ref, q_ref, k_ref, v_ref, xr_ref, *, rt, scale):
    C = x_ref.shape[-1]
    rs = x_ref.shape[1]
    xt = x_ref[...].reshape(2 * rt, C)
    inv = jnp.float32(1.0 / rt)
    xr_ref[...] = (jnp.sum(x_ref[0], axis=(0, 2)) * inv).reshape(2, 1, C)
    nn = (((1,), (0,)), ((), ()))
    x_hi = xt.astype(jnp.bfloat16)
    x_lo = (xt - x_hi.astype(jnp.float32)).astype(jnp.bfloat16)
    qkv = (jax.lax.dot_general(x_hi, w_ref[...], nn,
                               preferred_element_type=jnp.float32) +
           jax.lax.dot_general(x_lo, w_ref[...], nn,
                               preferred_element_type=jnp.float32))
    q = (qkv[:, :C] + b_ref[0:1, :]) * scale
    k = qkv[:, C:2 * C] + b_ref[1:2, :]
    v = qkv[:, 2 * C:] + b_ref[2:3, :]
    q_ref[...] = q.astype(jnp.bfloat16).reshape(1, rs, 2, rs, C)
    k_ref[...] = k.astype(jnp.bfloat16).reshape(1, rs, 2, rs, C)
    v_ref[...] = v.astype(jnp.bfloat16).reshape(1, rs, 2, rs, C)


def _routing_body(xr_ref, wq_ref, wk_ref, b_ref, a_ref):
    xr = xr_ref[...]
    qr = jnp.dot(xr, wq_ref[...], preferred_element_type=jnp.float32) + b_ref[0:1, :]
    kr = jnp.dot(xr, wk_ref[...], preferred_element_type=jnp.float32) + b_ref[1:2, :]
    a = jax.lax.dot_general(qr, kr, (((1,), (1,)), ((), ())),
                            preferred_element_type=jnp.float32)
    pad = a_ref.shape[1] - a.shape[1]
    a_ref[...] = jnp.concatenate(
        [a, jnp.full((a.shape[0], pad), -jnp.inf, jnp.float32)], axis=1)


def _make_sc_topk(R2, padc):
    n_work = 28
    rows_per = R2 // n_work
    chunks = padc // 16
    mesh = plsc.VectorSubcoreMesh(core_axis_name="c", subcore_axis_name="s")

    @functools.partial(
        pl.kernel, mesh=mesh,
        out_type=jax.ShapeDtypeStruct((R2, 16), jnp.int32),
        scratch_types=[pltpu.VMEM((padc,), jnp.float32),
                       pltpu.VMEM((16,), jnp.int32)],
    )
    def _sc_topk(a_hbm, idx_hbm, row_v, idx_v):
        wid = jax.lax.axis_index("s") * 2 + jax.lax.axis_index("c")

        @pl.when(wid < n_work)
        def _():
            lanes = jax.lax.iota(jnp.int32, 16)
            big = jnp.int32(1 << 30)
            neg = jnp.float32(-jnp.inf)
            mask15 = jnp.int32(15)

            dnums = jax.lax.GatherDimensionNumbers(
                offset_dims=(), collapsed_slice_dims=(0,), start_index_map=(0,))

            def shuf(v, perm):
                return jax.lax.gather(
                    v, perm[:, None], dnums, (1,),
                    mode=jax.lax.GatherScatterMode.PROMISE_IN_BOUNDS)

            def allmax(v):
                for sh in (1, 2, 4, 8):
                    v = jnp.maximum(v, shuf(v, (lanes + sh) & mask15))
                return v

            def allmin(v):
                for sh in (1, 2, 4, 8):
                    v = jnp.minimum(v, shuf(v, (lanes + sh) & mask15))
                return v

            for t in range(rows_per):
                r = wid * rows_per + t
                pltpu.sync_copy(a_hbm.at[r], row_v)
                m = jnp.full((16,), neg, jnp.float32)
                for c in range(chunks):
                    m = jnp.maximum(m, row_v[pl.ds(c * 16, 16)])
                m1 = allmax(m)
                best = jnp.full((16,), big, jnp.int32)
                for c in range(chunks):
                    ch = row_v[pl.ds(c * 16, 16)]
                    best = jnp.minimum(
                        best, jnp.where(ch == m1, lanes + c * 16, big))
                i1 = allmin(best)
                m2v = jnp.full((16,), neg, jnp.float32)
                for c in range(chunks):
                    ch = row_v[pl.ds(c * 16, 16)]
                    pos = lanes + c * 16
                    m2v = jnp.maximum(m2v, jnp.where(pos == i1, neg, ch))
                m2 = allmax(m2v)
                best2 = jnp.full((16,), big, jnp.int32)
                for c in range(chunks):
                    ch = row_v[pl.ds(c * 16, 16)]
                    pos = lanes + c * 16
                    best2 = jnp.minimum(
                        best2, jnp.where((ch == m2) & (pos != i1), pos, big))
                i2 = allmin(best2)
                idx_v[...] = jnp.where(lanes == 0, i1,
                                       jnp.where(lanes == 1, i2, 0))
                pltpu.sync_copy(idx_v, idx_hbm.at[r])

    return _sc_topk


def _attn_pair_body(idx_ref, q_ref, k0a_ref, k1a_ref, k0b_ref, k1b_ref,
                    v0a_ref, v1a_ref, v0b_ref, v1b_ref, o_ref, *, rt, nh):
    del idx_ref
    C = q_ref.shape[-1]
    rs = q_ref.shape[1]
    hd = C // nh
    nt = (((1,), (1,)), ((), ()))
    nn = (((1,), (0,)), ((), ()))
    for j, (k0_ref, k1_ref, v0_ref, v1_ref) in enumerate(
            [(k0a_ref, k1a_ref, v0a_ref, v1a_ref),
             (k0b_ref, k1b_ref, v0b_ref, v1b_ref)]):
        q = q_ref[0, :, j, :, :].reshape(rt, C)
        k0 = k0_ref[...].reshape(rt, C)
        k1 = k1_ref[...].reshape(rt, C)
        v0 = v0_ref[...].reshape(rt, C)
        v1 = v1_ref[...].reshape(rt, C)
        outs = []
        for h in range(nh):
            sl = slice(h * hd, (h + 1) * hd)
            qh = q[:, sl]
            e0 = jnp.exp(jax.lax.dot_general(
                qh, k0[:, sl], nt, preferred_element_type=jnp.float32))
            e1 = jnp.exp(jax.lax.dot_general(
                qh, k1[:, sl], nt, preferred_element_type=jnp.float32))
            den = (jnp.sum(e0, axis=1, keepdims=True) +
                   jnp.sum(e1, axis=1, keepdims=True))
            o = (jax.lax.dot_general(e0.astype(jnp.bfloat16), v0[:, sl], nn,
                                     preferred_element_type=jnp.float32) +
                 jax.lax.dot_general(e1.astype(jnp.bfloat16), v1[:, sl], nn,
                                     preferred_element_type=jnp.float32))
            outs.append(o / den)
        o = jnp.concatenate(outs, axis=1).astype(jnp.bfloat16)
        o_ref[0, :, j, :, :] = o.reshape(rs, rs, C)


def _final_body(attn_ref, vc_ref, vp_ref, vn_ref, lw_ref, lb_ref, ow_ref, ob_ref,
                o_ref, sh_ref, *, R, Wd):
    i = pl.program_id(0)
    n = pl.num_programs(0)
    C = vc_ref.shape[-1]
    pe = SIDE // 2
    f32 = jnp.float32
    top = vp_ref[...].astype(f32) * jnp.where(i == 0, 0.0, 1.0)
    bot = vn_ref[...].astype(f32) * jnp.where(i == n - 1, 0.0, 1.0)
    rows = jnp.concatenate([top, vc_ref[...].astype(f32), bot], axis=0)
    zc = jnp.zeros((R + 2 * pe, pe, C), f32)
    padded = jnp.concatenate([zc, rows, zc], axis=1)
    for dx in range(SIDE):
        sh_ref[dx] = padded[:, dx:dx + Wd, :]
    acc = attn_ref[...].astype(f32) + lb_ref[...]
    for dy in range(SIDE):
        for dx in range(SIDE):
            w = lw_ref[dy * SIDE + dx:dy * SIDE + dx + 1, :]
            acc = acc + sh_ref[dx, dy:dy + R] * w
    a2 = acc.reshape(R * Wd, C)
    a_hi = a2.astype(jnp.bfloat16)
    a_lo = (a2 - a_hi.astype(f32)).astype(jnp.bfloat16)
    y = (jnp.dot(a_hi, ow_ref[...], preferred_element_type=f32) +
         jnp.dot(a_lo, ow_ref[...], preferred_element_type=f32)) + ob_ref[...]
    o_ref[...] = y


def kernel(x, qkv_w, qkv_b, lepe_w, lepe_b, out_w, out_b):
    _, C, Hh, Ww = x.shape
    rs = max(1, Hh // N_WIN)
    rh, rw = Hh // rs, Ww // rs
    R2 = rh * rw
    rt = rs * rs
    HW = Hh * Ww
    scale = C ** (-0.5)
    f32 = jnp.float32
    bf16 = jnp.bfloat16

    x5 = x.reshape(C, HW).T.reshape(rh, rs, rw, rs, C)
    w_all = qkv_w.T.astype(bf16)
    wq, wk = qkv_w[0:C].T, qkv_w[C:2 * C].T
    b3 = qkv_b.reshape(3, C)

    blk5 = (1, rs, 1, rs, C)
    blk5q = (1, rs, 2, rs, C)
    imap5q = lambda g: (g // (rw // 2), 0, g % (rw // 2), 0, 0)
    cmap2 = lambda i: (0, 0)

    q5, k5, v5, xr = pl.pallas_call(
        functools.partial(_qkvpool_body, rt=rt, scale=scale),
        grid=(R2 // 2,),
        in_specs=[pl.BlockSpec(blk5q, imap5q),
                  pl.BlockSpec((C, 3 * C), cmap2),
                  pl.BlockSpec((3, C), cmap2)],
        out_specs=[pl.BlockSpec(blk5q, imap5q)] * 3 +
                  [pl.BlockSpec((2, 1, C), lambda g: (g, 0, 0))],
        out_shape=[jax.ShapeDtypeStruct((rh, rs, rw, rs, C), bf16)] * 3 +
                  [jax.ShapeDtypeStruct((R2, 1, C), f32)],
    )(x5, w_all, b3)

    padc = ((R2 + 15) // 16) * 16
    a_pad = pl.pallas_call(
        _routing_body,
        out_shape=jax.ShapeDtypeStruct((R2, padc), f32),
    )(xr.reshape(R2, C), wq, wk, b3)
    idx16 = _make_sc_topk(R2, padc)(a_pad)
    idx_flat = idx16[:, :TOPK].reshape(-1)

    rw2 = rw // 2
    blk5p = (1, rs, 2, rs, C)
    qmap = lambda g, idx: (g // rw2, 0, g % rw2, 0, 0)

    def gmap(o):
        return lambda g, idx: (idx[4 * g + o] // rw, 0, idx[4 * g + o] % rw, 0, 0)

    attn5 = pl.pallas_call(
        functools.partial(_attn_pair_body, rt=rt, nh=NUM_HEADS),
        grid_spec=pltpu.PrefetchScalarGridSpec(
            num_scalar_prefetch=1,
            grid=(R2 // 2,),
            in_specs=[pl.BlockSpec(blk5p, qmap)] +
                     [pl.BlockSpec(blk5, gmap(o)) for o in (0, 1, 2, 3)] +
                     [pl.BlockSpec(blk5, gmap(o)) for o in (0, 1, 2, 3)],
            out_specs=pl.BlockSpec(blk5p, qmap),
        ),
        out_shape=jax.ShapeDtypeStruct((rh, rs, rw, rs, C), bf16),
    )(idx_flat, q5, k5, k5, k5, k5, v5, v5, v5, v5)

    R = 4
    attn3 = attn5.reshape(Hh, Ww, C)
    v3 = v5.reshape(Hh, Ww, C)
    lw = lepe_w.reshape(C, SIDE * SIDE).T
    out_t = pl.pallas_call(
        functools.partial(_final_body, R=R, Wd=Ww),
        grid=(Hh // R,),
        in_specs=[pl.BlockSpec((R, Ww, C), lambda i: (i, 0, 0)),
                  pl.BlockSpec((R, Ww, C), lambda i: (i, 0, 0)),
                  pl.BlockSpec((2, Ww, C),
                               lambda i: (jnp.maximum(2 * i - 1, 0), 0, 0)),
                  pl.BlockSpec((2, Ww, C),
                               lambda i: (jnp.minimum(2 * i + 2, Hh // 2 - 1), 0, 0)),
                  pl.BlockSpec((SIDE * SIDE, C), cmap2),
                  pl.BlockSpec((1, C), cmap2),
                  pl.BlockSpec((C, C), cmap2),
                  pl.BlockSpec((1, C), cmap2)],
        out_specs=pl.BlockSpec((R * Ww, C), lambda i: (i, 0)),
        out_shape=jax.ShapeDtypeStruct((HW, C), f32),
        scratch_shapes=[pltpu.VMEM((SIDE, R + 2 * (SIDE // 2), Ww, C), f32)],
    )(attn3, v3, v3, v3, lw, lepe_b.reshape(1, C),
      out_w.T.astype(bf16), out_b.reshape(1, C))

    return out_t.T.reshape(1, C, Hh, Ww)
